# baseline (speedup 1.0000x reference)
import functools

import jax
import jax.numpy as jnp
from jax import lax
from jax.experimental import pallas as pl
from jax.experimental.pallas import tpu as pltpu
from jax.experimental.pallas import tpu_sc as plsc


_N = 512
_K = 1024
_D = 256
_M = 4


def _tree_reduce_cols(sqt):
    halves = []
    for t in (0, 1):
        base = 128 * t
        acc = sqt[base:base + 8, :]
        for v in range(1, 16):
            acc = acc + sqt[base + 8 * v:base + 8 * v + 8, :]
        p = acc
        r = ((p[0:1, :] + p[4:5, :]) + (p[2:3, :] + p[6:7, :])) + (
            (p[1:2, :] + p[5:6, :]) + (p[3:4, :] + p[7:8, :]))
        halves.append(r)
    return halves[0] + halves[1]


def _topm_kernel(z_ref, w_ref, idx_ref):
    z = z_ref[...]
    w = w_ref[...]
    wn = jnp.sum(w * w, axis=1, keepdims=True)
    wn_row = jnp.swapaxes(wn, 0, 1)
    z_hi = z.astype(jnp.bfloat16)
    z_mid = (z - z_hi.astype(jnp.float32)).astype(jnp.bfloat16)
    s_hi = w.astype(jnp.bfloat16)
    s_mid = (w - s_hi.astype(jnp.float32)).astype(jnp.bfloat16)
    dns = (((1,), (1,)), ((), ()))
    scores = (jax.lax.dot_general(z_hi, s_hi, dns,
                                  preferred_element_type=jnp.float32)
              + (jax.lax.dot_general(z_hi, s_mid, dns,
                                     preferred_element_type=jnp.float32)
                 + jax.lax.dot_general(z_mid, s_hi, dns,
                                       preferred_element_type=jnp.float32)))
    da = wn_row - 2.0 * scores

    iota_k = jax.lax.broadcasted_iota(jnp.int32, (_N, _K), 1)
    for j in range(_M):
        dmin = jnp.min(da, axis=1, keepdims=True)
        idx_j = jnp.min(jnp.where(da == dmin, iota_k, _K),
                        axis=1, keepdims=True)
        idx_ref[pl.ds(j * _N, _N), :] = idx_j
        if j < _M - 1:
            da = jnp.where(iota_k == idx_j, jnp.float32(3e38), da)


def _recheck_kernel(z_ref, g_ref, idx_ref, oidx_ref, zq_ref, ma_ref):
    z = z_ref[...]
    gt = jnp.swapaxes(g_ref[...], 0, 1)
    zt = jnp.swapaxes(z, 0, 1)
    zt_rep = jnp.concatenate([zt] * _M, axis=1)
    diff = zt_rep - gt
    dx = _tree_reduce_cols(diff * diff)

    best_d = dx[:, 0:_N]
    best_i = jnp.swapaxes(idx_ref[0:_N, :], 0, 1)
    best_g = gt[:, 0:_N]
    for j in range(1, _M):
        dj = dx[:, j * _N:(j + 1) * _N]
        ij = jnp.swapaxes(idx_ref[j * _N:(j + 1) * _N, :], 0, 1)
        better = (dj < best_d) | ((dj == best_d) & (ij < best_i))
        best_d = jnp.where(better, dj, best_d)
        best_i = jnp.where(better, ij, best_i)
        best_g = jnp.where(better, gt[:, j * _N:(j + 1) * _N], best_g)

    oidx_ref[...] = jnp.swapaxes(best_i, 0, 1)
    zq = jnp.swapaxes(best_g, 0, 1)
    zq_ref[...] = zq
    ma_ref[...] = z + (zq - z)


def _make_sc_gather():
    info = plsc.get_sparse_core_info()
    nw = info.num_cores * info.num_subcores
    b = _M * _N
    b_per_w = b // nw
    mesh = plsc.VectorSubcoreMesh(core_axis_name="c", subcore_axis_name="s")

    @functools.partial(
        pl.kernel, mesh=mesh,
        out_type=jax.ShapeDtypeStruct((b, _D), jnp.float32),
        scratch_types=[
            pltpu.VMEM((b_per_w,), jnp.int32),
            pltpu.VMEM((b_per_w, _D), jnp.float32),
            pltpu.SemaphoreType.DMA,
        ],
    )
    def sc_gather(table_hbm, idx_hbm, out_hbm, idx_v, rows_v, sem):
        wid = lax.axis_index("s") * info.num_cores + lax.axis_index("c")
        base = wid * b_per_w
        pltpu.sync_copy(idx_hbm.at[pl.ds(base, b_per_w)], idx_v)
        pltpu.async_copy(table_hbm.at[idx_v], rows_v, sem).wait()
        pltpu.sync_copy(rows_v, out_hbm.at[pl.ds(base, b_per_w)])

    return sc_gather


_sc_gather_cache = []


def _get_sc_gather():
    if not _sc_gather_cache:
        _sc_gather_cache.append(_make_sc_gather())
    return _sc_gather_cache[0]


def kernel(x, weight):
    z = jnp.transpose(x, (0, 2, 3, 1))
    zf = z.reshape(_N, _D)
    idx_cat = pl.pallas_call(
        _topm_kernel,
        out_shape=jax.ShapeDtypeStruct((_M * _N, 1), jnp.int32),
    )(zf, weight)
    g = _get_sc_gather()(weight, idx_cat.reshape(_M * _N))
    idx2, zqf, maf = pl.pallas_call(
        _recheck_kernel,
        out_shape=(
            jax.ShapeDtypeStruct((_N, 1), jnp.int32),
            jax.ShapeDtypeStruct((_N, _D), jnp.float32),
            jax.ShapeDtypeStruct((_N, _D), jnp.float32),
        ),
    )(zf, g, idx_cat)
    indices = idx2.reshape(_N)
    z_q = zqf.reshape(z.shape)
    z_q_ma = jnp.transpose(maf.reshape(z.shape), (0, 3, 1, 2))
    return (z_q_ma, z_q, z, indices)


# speedup vs baseline: 1.8257x; 1.8257x over previous
import jax
import jax.numpy as jnp
from jax.experimental import pallas as pl


_N = 512
_K = 1024
_D = 256
_M = 4

_HI = jax.lax.Precision.HIGHEST


def _tree_reduce_cols(sqt):
    halves = []
    for t in (0, 1):
        base = 128 * t
        acc = sqt[base:base + 8, :]
        for v in range(1, 16):
            acc = acc + sqt[base + 8 * v:base + 8 * v + 8, :]
        p = acc
        r = ((p[0:1, :] + p[4:5, :]) + (p[2:3, :] + p[6:7, :])) + (
            (p[1:2, :] + p[5:6, :]) + (p[3:4, :] + p[7:8, :]))
        halves.append(r)
    return halves[0] + halves[1]


def _vq_kernel(x_ref, w_ref, idx_ref, zq_ref, ma_ref, z_out_ref):
    xv = x_ref[...]
    xr = jnp.reshape(xv, (2, _D, _N // 2))
    z = jnp.concatenate(
        [jnp.swapaxes(xr[0], 0, 1), jnp.swapaxes(xr[1], 0, 1)], axis=0)
    z_out_ref[...] = z
    w = w_ref[...]

    wn = jnp.sum(w * w, axis=1, keepdims=True)
    wn_row = jnp.swapaxes(wn, 0, 1)
    z_hi = z.astype(jnp.bfloat16)
    z_mid = (z - z_hi.astype(jnp.float32)).astype(jnp.bfloat16)
    s_hi = w.astype(jnp.bfloat16)
    s_mid = (w - s_hi.astype(jnp.float32)).astype(jnp.bfloat16)
    dns = (((1,), (1,)), ((), ()))
    scores = (jax.lax.dot_general(z_hi, s_hi, dns,
                                  preferred_element_type=jnp.float32)
              + (jax.lax.dot_general(z_hi, s_mid, dns,
                                     preferred_element_type=jnp.float32)
                 + jax.lax.dot_general(z_mid, s_hi, dns,
                                       preferred_element_type=jnp.float32)))
    da = wn_row - 2.0 * scores

    iota_k = jax.lax.broadcasted_iota(jnp.int32, (_N, _K), 1)
    cand_cols = []
    for j in range(_M):
        dmin = jnp.min(da, axis=1, keepdims=True)
        idx_j = jnp.min(jnp.where(da == dmin, iota_k, _K),
                        axis=1, keepdims=True)
        cand_cols.append(idx_j)
        if j < _M - 1:
            da = jnp.where(iota_k == idx_j, jnp.float32(3e38), da)

    idx_cat = jnp.concatenate(cand_cols, axis=0)

    onehot = (jax.lax.broadcasted_iota(jnp.int32, (_M * _N, _K), 1)
              == idx_cat).astype(jnp.bfloat16)
    w_hi = w.astype(jnp.bfloat16)
    r1 = w - w_hi.astype(jnp.float32)
    w_mid = r1.astype(jnp.bfloat16)
    w_lo = (r1 - w_mid.astype(jnp.float32)).astype(jnp.bfloat16)
    dn = (((0,), (1,)), ((), ()))
    gt_hi = jax.lax.dot_general(w_hi, onehot, dn,
                                preferred_element_type=jnp.float32)
    gt_mid = jax.lax.dot_general(w_mid, onehot, dn,
                                 preferred_element_type=jnp.float32)
    gt_lo = jax.lax.dot_general(w_lo, onehot, dn,
                                preferred_element_type=jnp.float32)
    gt = (gt_hi + gt_mid) + gt_lo

    zt = jnp.swapaxes(z, 0, 1)
    zt_rep = jnp.concatenate([zt] * _M, axis=1)
    diff = zt_rep - gt
    dx = _tree_reduce_cols(diff * diff)

    best_d = dx[:, 0:_N]
    best_i = jnp.swapaxes(cand_cols[0], 0, 1)
    best_g = gt[:, 0:_N]
    for j in range(1, _M):
        dj = dx[:, j * _N:(j + 1) * _N]
        ij = jnp.swapaxes(cand_cols[j], 0, 1)
        better = (dj < best_d) | ((dj == best_d) & (ij < best_i))
        best_d = jnp.where(better, dj, best_d)
        best_i = jnp.where(better, ij, best_i)
        best_g = jnp.where(better, gt[:, j * _N:(j + 1) * _N], best_g)

    idx_ref[...] = jnp.swapaxes(best_i, 0, 1)
    zq = jnp.swapaxes(best_g, 0, 1)
    zq_ref[...] = zq
    ma_ref[...] = z + (zq - z)


def kernel(x, weight):
    idx2, zqf, maf, zf = pl.pallas_call(
        _vq_kernel,
        out_shape=(
            jax.ShapeDtypeStruct((_N, 1), jnp.int32),
            jax.ShapeDtypeStruct((_N, _D), jnp.float32),
            jax.ShapeDtypeStruct((_N, _D), jnp.float32),
            jax.ShapeDtypeStruct((_N, _D), jnp.float32),
        ),
    )(x, weight)
    indices = idx2.reshape(_N)
    z = zf.reshape(2, 16, 16, _D)
    z_q = zqf.reshape(z.shape)
    z_q_ma = jnp.transpose(maf.reshape(z.shape), (0, 3, 1, 2))
    return (z_q_ma, z_q, z, indices)


# speedup vs baseline: 2.5828x; 1.4147x over previous
import jax
import jax.numpy as jnp
from jax.experimental import pallas as pl


_N = 512
_K = 1024
_D = 256
_M = 4


def _tree_reduce_cols(sqt):
    halves = []
    for t in (0, 1):
        base = 128 * t
        acc = sqt[base:base + 8, :]
        for v in range(1, 16):
            acc = acc + sqt[base + 8 * v:base + 8 * v + 8, :]
        p = acc
        r = ((p[0:1, :] + p[4:5, :]) + (p[2:3, :] + p[6:7, :])) + (
            (p[1:2, :] + p[5:6, :]) + (p[3:4, :] + p[7:8, :]))
        halves.append(r)
    return halves[0] + halves[1]


def _vq_kernel(z_ref, w_ref, idx_ref, zq_ref, ma_ref):
    z = z_ref[...]
    w = w_ref[...]

    wn = jnp.sum(w * w, axis=1, keepdims=True)
    wn_row = jnp.swapaxes(wn, 0, 1)
    z_hi = z.astype(jnp.bfloat16)
    z_mid = (z - z_hi.astype(jnp.float32)).astype(jnp.bfloat16)
    s_hi = w.astype(jnp.bfloat16)
    s_mid = (w - s_hi.astype(jnp.float32)).astype(jnp.bfloat16)
    dns = (((1,), (1,)), ((), ()))
    scores = (jax.lax.dot_general(z_hi, s_hi, dns,
                                  preferred_element_type=jnp.float32)
              + (jax.lax.dot_general(z_hi, s_mid, dns,
                                     preferred_element_type=jnp.float32)
                 + jax.lax.dot_general(z_mid, s_hi, dns,
                                       preferred_element_type=jnp.float32)))
    da = wn_row - 2.0 * scores

    iota_k = jax.lax.broadcasted_iota(jnp.int32, (_N, _K), 1)
    cand_cols = []
    for j in range(_M):
        dmin = jnp.min(da, axis=1, keepdims=True)
        idx_j = jnp.min(jnp.where(da == dmin, iota_k, _K),
                        axis=1, keepdims=True)
        cand_cols.append(idx_j)
        if j < _M - 1:
            da = jnp.where(iota_k == idx_j, jnp.float32(3e38), da)

    idx_cat = jnp.concatenate(cand_cols, axis=0)

    onehot = (jax.lax.broadcasted_iota(jnp.int32, (_M * _N, _K), 1)
              == idx_cat).astype(jnp.bfloat16)
    w_hi = w.astype(jnp.bfloat16)
    r1 = w - w_hi.astype(jnp.float32)
    w_mid = r1.astype(jnp.bfloat16)
    w_lo = (r1 - w_mid.astype(jnp.float32)).astype(jnp.bfloat16)
    dn = (((0,), (1,)), ((), ()))
    gt_hi = jax.lax.dot_general(w_hi, onehot, dn,
                                preferred_element_type=jnp.float32)
    gt_mid = jax.lax.dot_general(w_mid, onehot, dn,
                                 preferred_element_type=jnp.float32)
    gt_lo = jax.lax.dot_general(w_lo, onehot, dn,
                                preferred_element_type=jnp.float32)
    gt = (gt_hi + gt_mid) + gt_lo

    zt = jnp.swapaxes(z, 0, 1)
    zt_rep = jnp.concatenate([zt] * _M, axis=1)
    diff = zt_rep - gt
    dx = _tree_reduce_cols(diff * diff)

    best_d = dx[:, 0:_N]
    best_i = jnp.swapaxes(cand_cols[0], 0, 1)
    best_g = gt[:, 0:_N]
    for j in range(1, _M):
        dj = dx[:, j * _N:(j + 1) * _N]
        ij = jnp.swapaxes(cand_cols[j], 0, 1)
        better = (dj < best_d) | ((dj == best_d) & (ij < best_i))
        best_d = jnp.where(better, dj, best_d)
        best_i = jnp.where(better, ij, best_i)
        best_g = jnp.where(better, gt[:, j * _N:(j + 1) * _N], best_g)

    idx_ref[...] = jnp.swapaxes(best_i, 0, 1)
    zq = jnp.swapaxes(best_g, 0, 1)
    zq_ref[...] = zq
    ma_ref[...] = z + (zq - z)


def kernel(x, weight):
    z = jnp.transpose(x, (0, 2, 3, 1))
    zf = z.reshape(_N, _D)
    idx2, zqf, maf = pl.pallas_call(
        _vq_kernel,
        out_shape=(
            jax.ShapeDtypeStruct((_N, 1), jnp.int32),
            jax.ShapeDtypeStruct((_N, _D), jnp.float32),
            jax.ShapeDtypeStruct((_N, _D), jnp.float32),
        ),
    )(zf, weight)
    indices = idx2.reshape(_N)
    z_q = zqf.reshape(z.shape)
    z_q_ma = jnp.transpose(maf.reshape(z.shape), (0, 3, 1, 2))
    return (z_q_ma, z_q, z, indices)


# speedup vs baseline: 2.7369x; 1.0597x over previous
import jax
import jax.numpy as jnp
from jax.experimental import pallas as pl


_N = 512
_K = 1024
_D = 256
_M = 4


def _tree_reduce_cols(sqt):
    halves = []
    for t in (0, 1):
        base = 128 * t
        acc = sqt[base:base + 8, :]
        for v in range(1, 16):
            acc = acc + sqt[base + 8 * v:base + 8 * v + 8, :]
        p = acc
        r = ((p[0:1, :] + p[4:5, :]) + (p[2:3, :] + p[6:7, :])) + (
            (p[1:2, :] + p[5:6, :]) + (p[3:4, :] + p[7:8, :]))
        halves.append(r)
    return halves[0] + halves[1]


def _vq_kernel(z_ref, w_ref, idx_ref, zq_ref, ma_ref):
    z = z_ref[...]
    w = w_ref[...]

    wn = jnp.sum(w * w, axis=1, keepdims=True)
    wn_row = jnp.swapaxes(wn, 0, 1)
    z_hi = z.astype(jnp.bfloat16)
    z_mid = (z - z_hi.astype(jnp.float32)).astype(jnp.bfloat16)
    s_hi = w.astype(jnp.bfloat16)
    s_mid = (w - s_hi.astype(jnp.float32)).astype(jnp.bfloat16)
    dns = (((1,), (1,)), ((), ()))
    scores = (jax.lax.dot_general(z_hi, s_hi, dns,
                                  preferred_element_type=jnp.float32)
              + (jax.lax.dot_general(z_hi, s_mid, dns,
                                     preferred_element_type=jnp.float32)
                 + jax.lax.dot_general(z_mid, s_hi, dns,
                                       preferred_element_type=jnp.float32)))
    da = wn_row - 2.0 * scores

    iota_k = jax.lax.broadcasted_iota(jnp.int32, (_N, _K), 1)
    bits = jax.lax.bitcast_convert_type(da, jnp.int32)
    mono = bits ^ ((bits >> 31) & jnp.int32(0x7FFFFFFF))
    keys = (mono & jnp.int32(~1023)) | iota_k
    cand_cols = []
    for j in range(_M):
        kmin = jnp.min(keys, axis=1, keepdims=True)
        cand_cols.append(kmin & jnp.int32(1023))
        if j < _M - 1:
            keys = jnp.where(keys == kmin, jnp.int32(0x7FFFFFFF), keys)

    idx_cat = jnp.concatenate(cand_cols, axis=0)

    onehot = (jax.lax.broadcasted_iota(jnp.int32, (_M * _N, _K), 1)
              == idx_cat).astype(jnp.bfloat16)
    w_hi = w.astype(jnp.bfloat16)
    r1 = w - w_hi.astype(jnp.float32)
    w_mid = r1.astype(jnp.bfloat16)
    w_lo = (r1 - w_mid.astype(jnp.float32)).astype(jnp.bfloat16)
    dn = (((0,), (1,)), ((), ()))
    gt_hi = jax.lax.dot_general(w_hi, onehot, dn,
                                preferred_element_type=jnp.float32)
    gt_mid = jax.lax.dot_general(w_mid, onehot, dn,
                                 preferred_element_type=jnp.float32)
    gt_lo = jax.lax.dot_general(w_lo, onehot, dn,
                                preferred_element_type=jnp.float32)
    gt = (gt_hi + gt_mid) + gt_lo

    zt = jnp.swapaxes(z, 0, 1)
    zt_rep = jnp.concatenate([zt] * _M, axis=1)
    diff = zt_rep - gt
    dx = _tree_reduce_cols(diff * diff)

    best_d = dx[:, 0:_N]
    best_i = jnp.swapaxes(cand_cols[0], 0, 1)
    best_g = gt[:, 0:_N]
    for j in range(1, _M):
        dj = dx[:, j * _N:(j + 1) * _N]
        ij = jnp.swapaxes(cand_cols[j], 0, 1)
        better = (dj < best_d) | ((dj == best_d) & (ij < best_i))
        best_d = jnp.where(better, dj, best_d)
        best_i = jnp.where(better, ij, best_i)
        best_g = jnp.where(better, gt[:, j * _N:(j + 1) * _N], best_g)

    idx_ref[...] = jnp.swapaxes(best_i, 0, 1)
    zq = jnp.swapaxes(best_g, 0, 1)
    zq_ref[...] = zq
    ma_ref[...] = z + (zq - z)


def kernel(x, weight):
    z = jnp.transpose(x, (0, 2, 3, 1))
    zf = z.reshape(_N, _D)
    idx2, zqf, maf = pl.pallas_call(
        _vq_kernel,
        out_shape=(
            jax.ShapeDtypeStruct((_N, 1), jnp.int32),
            jax.ShapeDtypeStruct((_N, _D), jnp.float32),
            jax.ShapeDtypeStruct((_N, _D), jnp.float32),
        ),
    )(zf, weight)
    indices = idx2.reshape(_N)
    z_q = zqf.reshape(z.shape)
    z_q_ma = jnp.transpose(maf.reshape(z.shape), (0, 3, 1, 2))
    return (z_q_ma, z_q, z, indices)
